# baseline (speedup 1.0000x reference)
import functools

import jax
import jax.numpy as jnp
from jax import lax
from jax.experimental import pallas as pl
from jax.experimental.pallas import tpu as pltpu
from jax.experimental.pallas import tpu_sc as plsc


_T = 256
_FT = 512
_DT = 256


def _sc_gather_rows(table, idx, chunk):
    rows, d = idx.shape[0], table.shape[1]
    info = plsc.get_sparse_core_info()
    nw = info.num_cores * info.num_subcores
    per_w = rows // nw
    assert per_w * nw == rows and per_w % chunk == 0 and per_w % 8 == 0
    n_ch = per_w // chunk
    mesh = plsc.VectorSubcoreMesh(core_axis_name="c", subcore_axis_name="s")

    @functools.partial(
        pl.kernel,
        mesh=mesh,
        out_type=jax.ShapeDtypeStruct((rows, d), table.dtype),
        scratch_types=[
            pltpu.VMEM((chunk,), jnp.int32),
            pltpu.VMEM((chunk, d), table.dtype),
            pltpu.SemaphoreType.DMA,
        ],
    )
    def gather_k(table_hbm, idx_hbm, out_hbm, idx_v, rows_v, sem):
        wid = lax.axis_index("s") * info.num_cores + lax.axis_index("c")
        base = wid * per_w
        for c in range(n_ch):
            off = base + c * chunk
            pltpu.sync_copy(idx_hbm.at[pl.ds(off, chunk)], idx_v)
            pltpu.async_copy(table_hbm.at[idx_v], rows_v, sem).wait()
            pltpu.sync_copy(rows_v, out_hbm.at[pl.ds(off, chunk)])

    return gather_k(table, idx)


def _p1_body(kk_ref, x_ref, wgu_f, wuu_f, wgg_f, wug_f, h_ref, wg_b, wu_b):
    t = pl.program_id(1)
    kk = kk_ref[0]

    @pl.when((t == 0) & (kk > 0))
    def _cast_und():
        wg_b[...] = wgu_f[...].astype(jnp.bfloat16)
        wu_b[...] = wuu_f[...].astype(jnp.bfloat16)

    @pl.when(t == kk)
    def _cast_gen():
        wg_b[...] = wgg_f[...].astype(jnp.bfloat16)
        wu_b[...] = wug_f[...].astype(jnp.bfloat16)

    g = jnp.dot(x_ref[...], wg_b[...], preferred_element_type=jnp.float32)
    u = jnp.dot(x_ref[...], wu_b[...], preferred_element_type=jnp.float32)
    h_ref[...] = (jax.nn.silu(g) * u).astype(jnp.bfloat16)


def _p2_body(kk_ref, h_ref, wdu_f, wdg_f, y_ref, wd_b):
    t = pl.program_id(1)
    kk = kk_ref[0]

    @pl.when((t == 0) & (kk > 0))
    def _cast_und():
        wd_b[...] = wdu_f[...].astype(jnp.bfloat16)

    @pl.when(t == kk)
    def _cast_gen():
        wd_b[...] = wdg_f[...].astype(jnp.bfloat16)

    y_ref[...] = jnp.dot(h_ref[...], wd_b[...],
                         preferred_element_type=jnp.float32)


def _moe_tc(kk_arr, xp_bf, wg_und, wu_und, wd_und, wg_gen, wu_gen, wd_gen):
    lp, d = xp_bf.shape
    f_dim = wg_und.shape[1]
    nf = (f_dim + _FT - 1) // _FT
    nt = lp // _T
    nd = d // _DT

    h = pl.pallas_call(
        _p1_body,
        grid_spec=pltpu.PrefetchScalarGridSpec(
            num_scalar_prefetch=1,
            grid=(nf, nt),
            in_specs=[
                pl.BlockSpec((_T, d), lambda f, t, kk: (t, 0)),
                pl.BlockSpec((d, _FT), lambda f, t, kk: (0, f)),
                pl.BlockSpec((d, _FT), lambda f, t, kk: (0, f)),
                pl.BlockSpec((d, _FT), lambda f, t, kk: (0, f)),
                pl.BlockSpec((d, _FT), lambda f, t, kk: (0, f)),
            ],
            out_specs=pl.BlockSpec((_T, _FT), lambda f, t, kk: (t, f)),
            scratch_shapes=[pltpu.VMEM((d, _FT), jnp.bfloat16)] * 2,
        ),
        out_shape=jax.ShapeDtypeStruct((lp, f_dim), jnp.bfloat16),
        compiler_params=pltpu.CompilerParams(
            dimension_semantics=("arbitrary", "arbitrary")),
    )(kk_arr, xp_bf, wg_und, wu_und, wg_gen, wu_gen)

    y = pl.pallas_call(
        _p2_body,
        grid_spec=pltpu.PrefetchScalarGridSpec(
            num_scalar_prefetch=1,
            grid=(nd, nt),
            in_specs=[
                pl.BlockSpec((_T, f_dim), lambda dd, t, kk: (t, 0)),
                pl.BlockSpec((f_dim, _DT), lambda dd, t, kk: (0, dd)),
                pl.BlockSpec((f_dim, _DT), lambda dd, t, kk: (0, dd)),
            ],
            out_specs=pl.BlockSpec((_T, _DT), lambda dd, t, kk: (t, dd)),
            scratch_shapes=[pltpu.VMEM((f_dim, _DT), jnp.bfloat16)],
        ),
        out_shape=jax.ShapeDtypeStruct((lp, d), jnp.float32),
        compiler_params=pltpu.CompilerParams(
            dimension_semantics=("arbitrary", "arbitrary")),
    )(kk_arr, h, wd_und, wd_gen)
    return y


def kernel(hidden_states, gen_token_mask, Wg_und, Wu_und, Wd_und, Wg_gen, Wu_gen, Wd_gen):
    b, l, d = hidden_states.shape
    n = b * l
    lp = n + _T
    x = hidden_states.reshape(n, d)
    m = gen_token_mask.reshape(n).astype(jnp.int32)

    n0 = n - jnp.sum(m)
    pad0 = ((n0 + _T - 1) // _T) * _T
    kk = pad0 // _T
    rank0 = jnp.cumsum(1 - m) - 1
    rank1 = jnp.cumsum(m) - 1
    loc = jnp.where(m > 0, pad0 + rank1, rank0).astype(jnp.int32)
    perm = jnp.zeros((lp,), jnp.int32).at[loc].set(
        jnp.arange(n, dtype=jnp.int32), mode="drop")

    x_bits = lax.bitcast_convert_type(
        x.astype(jnp.bfloat16).reshape(n, d // 2, 2), jnp.int32)
    xp_bits = _sc_gather_rows(x_bits, perm, chunk=24)
    xp_bf = lax.bitcast_convert_type(xp_bits, jnp.bfloat16).reshape(lp, d)

    y = _moe_tc(kk.reshape(1).astype(jnp.int32), xp_bf,
                Wg_und, Wu_und, Wd_und, Wg_gen, Wu_gen, Wd_gen)
    out = _sc_gather_rows(y, loc, chunk=16)
    return out.reshape(b, l, d)


# speedup vs baseline: 1.0475x; 1.0475x over previous
import functools

import jax
import jax.numpy as jnp
from jax import lax
from jax.experimental import pallas as pl
from jax.experimental.pallas import tpu as pltpu
from jax.experimental.pallas import tpu_sc as plsc


_T = 512
_FT = 512
_DT = 256


def _sc_info():
    info = plsc.get_sparse_core_info()
    return info, plsc.VectorSubcoreMesh(core_axis_name="c", subcore_axis_name="s")


def _sc_scatter_rows(src, idx, out_rows, chunk=16):
    rows, d = idx.shape[0], src.shape[1]
    info, mesh = _sc_info()
    nw = info.num_cores * info.num_subcores
    per_w = rows // nw
    assert per_w * nw == rows and per_w % chunk == 0 and per_w % 8 == 0
    n_ch = per_w // chunk

    @functools.partial(
        pl.kernel,
        mesh=mesh,
        out_type=jax.ShapeDtypeStruct((out_rows, d), src.dtype),
        scratch_types=[
            pltpu.VMEM((chunk,), jnp.int32),
            pltpu.VMEM((chunk, d), src.dtype),
            pltpu.SemaphoreType.DMA,
        ],
    )
    def scatter_k(src_hbm, idx_hbm, out_hbm, idx_v, rows_v, sem):
        wid = lax.axis_index("s") * info.num_cores + lax.axis_index("c")
        base = wid * per_w
        for c in range(n_ch):
            off = base + c * chunk
            pltpu.sync_copy(idx_hbm.at[pl.ds(off, chunk)], idx_v)
            pltpu.sync_copy(src_hbm.at[pl.ds(off, chunk)], rows_v)
            pltpu.async_copy(rows_v, out_hbm.at[idx_v], sem).wait()

    return scatter_k(src, idx)


def _sc_gather_rows(table, idx, chunk=16):
    rows, d = idx.shape[0], table.shape[1]
    info, mesh = _sc_info()
    nw = info.num_cores * info.num_subcores
    per_w = rows // nw
    assert per_w * nw == rows and per_w % chunk == 0 and per_w % 8 == 0
    n_ch = per_w // chunk

    @functools.partial(
        pl.kernel,
        mesh=mesh,
        out_type=jax.ShapeDtypeStruct((rows, d), table.dtype),
        scratch_types=[
            pltpu.VMEM((chunk,), jnp.int32),
            pltpu.VMEM((chunk, d), table.dtype),
            pltpu.SemaphoreType.DMA,
        ],
    )
    def gather_k(table_hbm, idx_hbm, out_hbm, idx_v, rows_v, sem):
        wid = lax.axis_index("s") * info.num_cores + lax.axis_index("c")
        base = wid * per_w
        for c in range(n_ch):
            off = base + c * chunk
            pltpu.sync_copy(idx_hbm.at[pl.ds(off, chunk)], idx_v)
            pltpu.async_copy(table_hbm.at[idx_v], rows_v, sem).wait()
            pltpu.sync_copy(rows_v, out_hbm.at[pl.ds(off, chunk)])

    return gather_k(table, idx)


def _p1_body(kk_ref, x_ref, wgu_f, wuu_f, wgg_f, wug_f, h_ref, wg_b, wu_b):
    t = pl.program_id(1)
    kk = kk_ref[0]

    @pl.when((t == 0) & (kk > 0))
    def _cast_und():
        wg_b[...] = wgu_f[...].astype(jnp.bfloat16)
        wu_b[...] = wuu_f[...].astype(jnp.bfloat16)

    @pl.when(t == kk)
    def _cast_gen():
        wg_b[...] = wgg_f[...].astype(jnp.bfloat16)
        wu_b[...] = wug_f[...].astype(jnp.bfloat16)

    g = jnp.dot(x_ref[...], wg_b[...], preferred_element_type=jnp.float32)
    u = jnp.dot(x_ref[...], wu_b[...], preferred_element_type=jnp.float32)
    h_ref[...] = (jax.nn.silu(g) * u).astype(jnp.bfloat16)


def _p2_body(kk_ref, h_ref, wdu_f, wdg_f, y_ref, wd_b):
    t = pl.program_id(1)
    kk = kk_ref[0]

    @pl.when((t == 0) & (kk > 0))
    def _cast_und():
        wd_b[...] = wdu_f[...].astype(jnp.bfloat16)

    @pl.when(t == kk)
    def _cast_gen():
        wd_b[...] = wdg_f[...].astype(jnp.bfloat16)

    y_ref[...] = jnp.dot(h_ref[...], wd_b[...],
                         preferred_element_type=jnp.float32)


def _moe_tc(kk_arr, xp_bf, wg_und, wu_und, wd_und, wg_gen, wu_gen, wd_gen):
    lp, d = xp_bf.shape
    f_dim = wg_und.shape[1]
    nf = (f_dim + _FT - 1) // _FT
    nt = lp // _T
    nd = d // _DT

    h = pl.pallas_call(
        _p1_body,
        grid_spec=pltpu.PrefetchScalarGridSpec(
            num_scalar_prefetch=1,
            grid=(nf, nt),
            in_specs=[
                pl.BlockSpec((_T, d), lambda f, t, kk: (t, 0)),
                pl.BlockSpec((d, _FT), lambda f, t, kk: (0, f)),
                pl.BlockSpec((d, _FT), lambda f, t, kk: (0, f)),
                pl.BlockSpec((d, _FT), lambda f, t, kk: (0, f)),
                pl.BlockSpec((d, _FT), lambda f, t, kk: (0, f)),
            ],
            out_specs=pl.BlockSpec((_T, _FT), lambda f, t, kk: (t, f)),
            scratch_shapes=[pltpu.VMEM((d, _FT), jnp.bfloat16)] * 2,
        ),
        out_shape=jax.ShapeDtypeStruct((lp, f_dim), jnp.bfloat16),
        compiler_params=pltpu.CompilerParams(
            dimension_semantics=("arbitrary", "arbitrary")),
    )(kk_arr, xp_bf, wg_und, wu_und, wg_gen, wu_gen)

    y = pl.pallas_call(
        _p2_body,
        grid_spec=pltpu.PrefetchScalarGridSpec(
            num_scalar_prefetch=1,
            grid=(nd, nt),
            in_specs=[
                pl.BlockSpec((_T, f_dim), lambda dd, t, kk: (t, 0)),
                pl.BlockSpec((f_dim, _DT), lambda dd, t, kk: (0, dd)),
                pl.BlockSpec((f_dim, _DT), lambda dd, t, kk: (0, dd)),
            ],
            out_specs=pl.BlockSpec((_T, _DT), lambda dd, t, kk: (t, dd)),
            scratch_shapes=[pltpu.VMEM((f_dim, _DT), jnp.bfloat16)],
        ),
        out_shape=jax.ShapeDtypeStruct((lp, d), jnp.float32),
        compiler_params=pltpu.CompilerParams(
            dimension_semantics=("arbitrary", "arbitrary")),
    )(kk_arr, h, wd_und, wd_gen)
    return y


def kernel(hidden_states, gen_token_mask, Wg_und, Wu_und, Wd_und, Wg_gen, Wu_gen, Wd_gen):
    b, l, d = hidden_states.shape
    n = b * l
    lp = n + _T
    x = hidden_states.reshape(n, d)
    m = gen_token_mask.reshape(n).astype(jnp.int32)

    n0 = n - jnp.sum(m)
    pad0 = ((n0 + _T - 1) // _T) * _T
    kk = pad0 // _T
    rank0 = jnp.cumsum(1 - m) - 1
    rank1 = jnp.cumsum(m) - 1
    loc = jnp.where(m > 0, pad0 + rank1, rank0).astype(jnp.int32)

    x_bits = lax.bitcast_convert_type(
        x.astype(jnp.bfloat16).reshape(n, d // 2, 2), jnp.int32)
    xp_bits = _sc_scatter_rows(x_bits, loc, lp)
    xp_bf = lax.bitcast_convert_type(xp_bits, jnp.bfloat16).reshape(lp, d)

    y = _moe_tc(kk.reshape(1).astype(jnp.int32), xp_bf,
                Wg_und, Wu_und, Wd_und, Wg_gen, Wu_gen, Wd_gen)
    out = _sc_gather_rows(y, loc)
    return out.reshape(b, l, d)


# speedup vs baseline: 1.5890x; 1.5169x over previous
import functools

import jax
import jax.numpy as jnp
from jax import lax
from jax.experimental import pallas as pl
from jax.experimental.pallas import tpu as pltpu
from jax.experimental.pallas import tpu_sc as plsc


_T = 512
_FT = 512
_DT = 256


def _sc_info():
    info = plsc.get_sparse_core_info()
    return info, plsc.VectorSubcoreMesh(core_axis_name="c", subcore_axis_name="s")


def _sc_scatter_rows(src, idx, out_rows, chunk=16):
    rows, d = idx.shape[0], src.shape[1]
    info, mesh = _sc_info()
    nw = info.num_cores * info.num_subcores
    per_w = rows // nw
    assert per_w * nw == rows and per_w % chunk == 0 and per_w % 8 == 0
    n_ch = per_w // chunk

    @functools.partial(
        pl.kernel,
        mesh=mesh,
        out_type=jax.ShapeDtypeStruct((out_rows, d), src.dtype),
        scratch_types=[
            pltpu.VMEM((chunk,), jnp.int32),
            pltpu.VMEM((chunk, d), src.dtype),
            pltpu.SemaphoreType.DMA,
        ],
    )
    def scatter_k(src_hbm, idx_hbm, out_hbm, idx_v, rows_v, sem):
        wid = lax.axis_index("s") * info.num_cores + lax.axis_index("c")
        base = wid * per_w
        for c in range(n_ch):
            off = base + c * chunk
            pltpu.sync_copy(idx_hbm.at[pl.ds(off, chunk)], idx_v)
            pltpu.sync_copy(src_hbm.at[pl.ds(off, chunk)], rows_v)
            pltpu.async_copy(rows_v, out_hbm.at[idx_v], sem).wait()

    return scatter_k(src, idx)


def _sc_gather_rows(table, idx, chunk=16):
    rows, d = idx.shape[0], table.shape[1]
    info, mesh = _sc_info()
    nw = info.num_cores * info.num_subcores
    per_w = rows // nw
    assert per_w * nw == rows and per_w % chunk == 0 and per_w % 8 == 0
    n_ch = per_w // chunk

    @functools.partial(
        pl.kernel,
        mesh=mesh,
        out_type=jax.ShapeDtypeStruct((rows, d), table.dtype),
        scratch_types=[
            pltpu.VMEM((chunk,), jnp.int32),
            pltpu.VMEM((chunk, d), table.dtype),
            pltpu.SemaphoreType.DMA,
        ],
    )
    def gather_k(table_hbm, idx_hbm, out_hbm, idx_v, rows_v, sem):
        wid = lax.axis_index("s") * info.num_cores + lax.axis_index("c")
        base = wid * per_w
        for c in range(n_ch):
            off = base + c * chunk
            pltpu.sync_copy(idx_hbm.at[pl.ds(off, chunk)], idx_v)
            pltpu.async_copy(table_hbm.at[idx_v], rows_v, sem).wait()
            pltpu.sync_copy(rows_v, out_hbm.at[pl.ds(off, chunk)])

    return gather_k(table, idx)


def _p1_body(kk_ref, x_ref, wgu_f, wuu_f, wgg_f, wug_f, h_ref, wg_b, wu_b):
    t = pl.program_id(1)
    kk = kk_ref[0]

    @pl.when((t == 0) & (kk > 0))
    def _cast_und():
        wg_b[...] = wgu_f[...].astype(jnp.bfloat16)
        wu_b[...] = wuu_f[...].astype(jnp.bfloat16)

    @pl.when(t == kk)
    def _cast_gen():
        wg_b[...] = wgg_f[...].astype(jnp.bfloat16)
        wu_b[...] = wug_f[...].astype(jnp.bfloat16)

    x_bf = x_ref[...].astype(jnp.bfloat16)
    g = jnp.dot(x_bf, wg_b[...], preferred_element_type=jnp.float32)
    u = jnp.dot(x_bf, wu_b[...], preferred_element_type=jnp.float32)
    h_ref[...] = (jax.nn.silu(g) * u).astype(jnp.bfloat16)


def _p2_body(kk_ref, h_ref, wdu_f, wdg_f, y_ref, wd_b):
    t = pl.program_id(1)
    kk = kk_ref[0]

    @pl.when((t == 0) & (kk > 0))
    def _cast_und():
        wd_b[...] = wdu_f[...].astype(jnp.bfloat16)

    @pl.when(t == kk)
    def _cast_gen():
        wd_b[...] = wdg_f[...].astype(jnp.bfloat16)

    y_ref[...] = jnp.dot(h_ref[...], wd_b[...],
                         preferred_element_type=jnp.float32)


def _moe_tc(kk_arr, xp_bf, wg_und, wu_und, wd_und, wg_gen, wu_gen, wd_gen):
    lp, d = xp_bf.shape
    f_dim = wg_und.shape[1]
    nf = (f_dim + _FT - 1) // _FT
    nt = lp // _T
    nd = d // _DT

    h = pl.pallas_call(
        _p1_body,
        grid_spec=pltpu.PrefetchScalarGridSpec(
            num_scalar_prefetch=1,
            grid=(nf, nt),
            in_specs=[
                pl.BlockSpec((_T, d), lambda f, t, kk: (t, 0)),
                pl.BlockSpec((d, _FT), lambda f, t, kk: (0, f)),
                pl.BlockSpec((d, _FT), lambda f, t, kk: (0, f)),
                pl.BlockSpec((d, _FT), lambda f, t, kk: (0, f)),
                pl.BlockSpec((d, _FT), lambda f, t, kk: (0, f)),
            ],
            out_specs=pl.BlockSpec((_T, _FT), lambda f, t, kk: (t, f)),
            scratch_shapes=[pltpu.VMEM((d, _FT), jnp.bfloat16)] * 2,
        ),
        out_shape=jax.ShapeDtypeStruct((lp, f_dim), jnp.bfloat16),
        compiler_params=pltpu.CompilerParams(
            dimension_semantics=("arbitrary", "arbitrary")),
    )(kk_arr, xp_bf, wg_und, wu_und, wg_gen, wu_gen)

    y = pl.pallas_call(
        _p2_body,
        grid_spec=pltpu.PrefetchScalarGridSpec(
            num_scalar_prefetch=1,
            grid=(nd, nt),
            in_specs=[
                pl.BlockSpec((_T, f_dim), lambda dd, t, kk: (t, 0)),
                pl.BlockSpec((f_dim, _DT), lambda dd, t, kk: (0, dd)),
                pl.BlockSpec((f_dim, _DT), lambda dd, t, kk: (0, dd)),
            ],
            out_specs=pl.BlockSpec((_T, _DT), lambda dd, t, kk: (t, dd)),
            scratch_shapes=[pltpu.VMEM((f_dim, _DT), jnp.bfloat16)],
        ),
        out_shape=jax.ShapeDtypeStruct((lp, d), jnp.float32),
        compiler_params=pltpu.CompilerParams(
            dimension_semantics=("arbitrary", "arbitrary")),
    )(kk_arr, h, wd_und, wd_gen)
    return y


def kernel(hidden_states, gen_token_mask, Wg_und, Wu_und, Wd_und, Wg_gen, Wu_gen, Wd_gen):
    b, l, d = hidden_states.shape
    n = b * l
    lp = n + _T
    x = hidden_states.reshape(n, d)
    m = gen_token_mask.reshape(n).astype(jnp.int32)

    n0 = n - jnp.sum(m)
    pad0 = ((n0 + _T - 1) // _T) * _T
    kk = pad0 // _T
    rank0 = jnp.cumsum(1 - m) - 1
    rank1 = jnp.cumsum(m) - 1
    loc = jnp.where(m > 0, pad0 + rank1, rank0).astype(jnp.int32)

    xp = _sc_scatter_rows(x, loc, lp)

    y = _moe_tc(kk.reshape(1).astype(jnp.int32), xp,
                Wg_und, Wu_und, Wd_und, Wg_gen, Wu_gen, Wd_gen)
    out = _sc_gather_rows(y, loc)
    return out.reshape(b, l, d)
